# Initial kernel scaffold; baseline (speedup 1.0000x reference)
import jax
import jax.numpy as jnp
from jax.experimental import pallas as pl


def kernel(logits, temperatures, top_ks, top_ps, min_ps):
    raise NotImplementedError("write your pallas kernel here")


# speedup vs baseline: 15.0623x; 15.0623x over previous
import jax
import jax.numpy as jnp
from jax.experimental import pallas as pl

_B = 128
_V = 100000
_ROWS = 8


def _sampler_kernel(logits_ref, g_ref, temp_ref, topk_ref, topp_ref,
                    minp_ref, out_ref):
    s = logits_ref[...] / temp_ref[...]
    m = jnp.max(s, axis=-1, keepdims=True)
    e = jnp.exp(s - m)
    ei = jax.lax.bitcast_convert_type(e, jnp.int32)
    k = topk_ref[...]

    def body1(i, t):
        cand = t + jnp.left_shift(1, 30 - i)
        cnt = jnp.sum((ei >= cand).astype(jnp.int32), axis=-1, keepdims=True)
        return jnp.where(cnt >= k, cand, t)

    t1 = jax.lax.fori_loop(0, 31, body1, jnp.zeros((_ROWS, 1), jnp.int32))

    e1 = jnp.where(ei >= t1, e, 0.0)
    p_target = topp_ref[...] * jnp.sum(e1, axis=-1, keepdims=True)

    def body2(i, t):
        cand = t + jnp.left_shift(1, 30 - i)
        ssum = jnp.sum(jnp.where(ei > cand, e1, 0.0), axis=-1, keepdims=True)
        return jnp.where(ssum >= p_target, cand, t)

    tlf = jax.lax.fori_loop(0, 31, body2, jnp.zeros((_ROWS, 1), jnp.int32))
    t2f = jax.lax.bitcast_convert_type(tlf + 1, jnp.float32)
    thresh = jnp.maximum(t2f, minp_ref[...])

    z = jnp.where(e >= thresh, s + g_ref[...], -jnp.inf)
    zmax = jnp.max(z, axis=-1, keepdims=True)
    iota = jax.lax.broadcasted_iota(jnp.int32, z.shape, 1)
    out_ref[...] = jnp.min(jnp.where(z == zmax, iota, _V), axis=-1,
                           keepdims=True)


@jax.jit
def kernel(logits, temperatures, top_ks, top_ps, min_ps):
    g = jax.random.gumbel(jax.random.key(42), (_B, _V), jnp.float32)
    row = lambda x: x.reshape(_B, 1)
    grid = (_B // _ROWS,)
    wide = pl.BlockSpec((_ROWS, _V), lambda i: (i, 0))
    slim = pl.BlockSpec((_ROWS, 1), lambda i: (i, 0))
    tokens = pl.pallas_call(
        _sampler_kernel,
        grid=grid,
        in_specs=[wide, wide, slim, slim, slim, slim],
        out_specs=slim,
        out_shape=jax.ShapeDtypeStruct((_B, 1), jnp.int32),
    )(logits, g, row(temperatures), row(top_ks.astype(jnp.int32)),
      row(top_ps), row(min_ps))
    return tokens.reshape(_B)


# speedup vs baseline: 15.4295x; 1.0244x over previous
import jax
import jax.numpy as jnp
from jax.experimental import pallas as pl
from jax.experimental.pallas import tpu as pltpu

_B = 128
_V = 100000
_ROWS = 8


def _sampler_kernel(logits_ref, g_ref, temp_ref, topk_ref, topp_ref,
                    minp_ref, out_ref):
    s = logits_ref[...] / temp_ref[...]
    m = jnp.max(s, axis=-1, keepdims=True)
    e = jnp.exp(s - m)
    ei = jax.lax.bitcast_convert_type(e, jnp.int32)
    k = topk_ref[...]

    def body1(i, t):
        cand = t + jnp.left_shift(1, 29 - i)
        cnt = jnp.sum((ei >= cand).astype(jnp.int32), axis=-1, keepdims=True)
        return jnp.where(cnt >= k, cand, t)

    t1 = jax.lax.fori_loop(0, 30, body1, jnp.zeros((_ROWS, 1), jnp.int32))

    e1 = jnp.where(ei >= t1, e, 0.0)
    p_target = topp_ref[...] * jnp.sum(e1, axis=-1, keepdims=True)

    def body2(i, t):
        cand = t + jnp.left_shift(1, 29 - i)
        ssum = jnp.sum(jnp.where(ei > cand, e1, 0.0), axis=-1, keepdims=True)
        return jnp.where(ssum >= p_target, cand, t)

    tlf = jax.lax.fori_loop(0, 30, body2, jnp.zeros((_ROWS, 1), jnp.int32))
    t2f = jax.lax.bitcast_convert_type(tlf + 1, jnp.float32)
    thresh = jnp.maximum(t2f, minp_ref[...])

    z = jnp.where(e >= thresh, s + g_ref[...], -jnp.inf)
    zmax = jnp.max(z, axis=-1, keepdims=True)
    iota = jax.lax.broadcasted_iota(jnp.int32, z.shape, 1)
    out_ref[...] = jnp.min(jnp.where(z == zmax, iota, _V), axis=-1,
                           keepdims=True)


@jax.jit
def kernel(logits, temperatures, top_ks, top_ps, min_ps):
    g = jax.random.gumbel(jax.random.key(42), (_B, _V), jnp.float32)
    row = lambda x: x.reshape(_B, 1)
    grid = (_B // _ROWS,)
    wide = pl.BlockSpec((_ROWS, _V), lambda i: (i, 0))
    slim = pl.BlockSpec((_ROWS, 1), lambda i: (i, 0))
    tokens = pl.pallas_call(
        _sampler_kernel,
        grid=grid,
        in_specs=[wide, wide, slim, slim, slim, slim],
        out_specs=slim,
        out_shape=jax.ShapeDtypeStruct((_B, 1), jnp.int32),
        compiler_params=pltpu.CompilerParams(
            dimension_semantics=("parallel",)),
    )(logits, g, row(temperatures), row(top_ks.astype(jnp.int32)),
      row(top_ps), row(min_ps))
    return tokens.reshape(_B)


# speedup vs baseline: 23.2364x; 1.5060x over previous
import functools

import jax
import jax.numpy as jnp
from jax.experimental import pallas as pl
from jax.experimental.pallas import tpu as pltpu

_B = 128
_V = 100000
_VP = 100352
_NS = 8
_SL = _VP // _NS
_ROWS = 8


def _sliced(x, red, comb):
    parts = [red(x[:, j * _SL:(j + 1) * _SL], axis=-1, keepdims=True)
             for j in range(_NS)]
    while len(parts) > 1:
        parts = [comb(parts[i], parts[i + 1])
                 for i in range(0, len(parts), 2)]
    return parts[0]


def _rsum(x):
    return _sliced(x, jnp.sum, jnp.add)


def _rmax(x):
    return _sliced(x, jnp.max, jnp.maximum)


def _rmin(x):
    return _sliced(x, jnp.min, jnp.minimum)


def _sampler_kernel(logits_ref, g_ref, temp_ref, topk_ref, topp_ref,
                    minp_ref, out_ref):
    s = logits_ref[...] / temp_ref[...]
    m = _rmax(s)
    e = jnp.exp(s - m)
    kf = topk_ref[...]

    def body1(i, t):
        cand = t + jnp.left_shift(1, 29 - i)
        candf = jax.lax.bitcast_convert_type(cand, jnp.float32)
        cnt = _rsum(jnp.where(e >= candf, 1.0, 0.0))
        return jnp.where(cnt >= kf, cand, t)

    t1 = jax.lax.fori_loop(0, 30, body1, jnp.zeros((_ROWS, 1), jnp.int32))
    t1f = jax.lax.bitcast_convert_type(t1, jnp.float32)

    e1 = jnp.where(e >= t1f, e, 0.0)
    p_target = topp_ref[...] * _rsum(e1)

    def body2(i, t):
        cand = t + jnp.left_shift(1, 29 - i)
        candf = jax.lax.bitcast_convert_type(cand, jnp.float32)
        ssum = _rsum(jnp.where(e1 > candf, e1, 0.0))
        return jnp.where(ssum >= p_target, cand, t)

    tlf = jax.lax.fori_loop(0, 30, body2, jnp.zeros((_ROWS, 1), jnp.int32))
    t2f = jax.lax.bitcast_convert_type(tlf + 1, jnp.float32)
    thresh = jnp.maximum(t2f, minp_ref[...])

    z = jnp.where(e1 >= thresh, s + g_ref[...], -jnp.inf)
    zmax = _rmax(z)
    iota = jax.lax.broadcasted_iota(jnp.int32, z.shape, 1)
    out_ref[...] = _rmin(jnp.where(z == zmax, iota, _V))


@jax.jit
def kernel(logits, temperatures, top_ks, top_ps, min_ps):
    g = jax.random.gumbel(jax.random.key(42), (_B, _V), jnp.float32)
    pad = ((0, 0), (0, _VP - _V))
    logits_p = jnp.pad(logits, pad, constant_values=-jnp.inf)
    g_p = jnp.pad(g, pad)
    row = lambda x: x.reshape(_B, 1)
    grid = (_B // _ROWS,)
    wide = pl.BlockSpec((_ROWS, _VP), lambda i: (i, 0))
    slim = pl.BlockSpec((_ROWS, 1), lambda i: (i, 0))
    tokens = pl.pallas_call(
        _sampler_kernel,
        grid=grid,
        in_specs=[wide, wide, slim, slim, slim, slim],
        out_specs=slim,
        out_shape=jax.ShapeDtypeStruct((_B, 1), jnp.int32),
        compiler_params=pltpu.CompilerParams(
            dimension_semantics=("parallel",)),
    )(logits_p, g_p, row(temperatures), row(top_ks.astype(jnp.float32)),
      row(top_ps), row(min_ps))
    return tokens.reshape(_B)


# speedup vs baseline: 24.1770x; 1.0405x over previous
import functools

import jax
import jax.numpy as jnp
from jax.experimental import pallas as pl
from jax.experimental.pallas import tpu as pltpu

_B = 128
_V = 100000
_VP = 100352
_NS = 16
_SL = _VP // _NS
_ROWS = 8


def _sliced(x, red, comb):
    parts = [red(x[:, j * _SL:(j + 1) * _SL], axis=-1, keepdims=True)
             for j in range(_NS)]
    while len(parts) > 1:
        parts = [comb(parts[i], parts[i + 1])
                 for i in range(0, len(parts), 2)]
    return parts[0]


def _rsum(x):
    return _sliced(x, jnp.sum, jnp.add)


def _rmax(x):
    return _sliced(x, jnp.max, jnp.maximum)


def _rmin(x):
    return _sliced(x, jnp.min, jnp.minimum)


def _sampler_kernel(logits_ref, g_ref, temp_ref, topk_ref, topp_ref,
                    minp_ref, out_ref):
    s = logits_ref[...] / temp_ref[...]
    m = _rmax(s)
    e = jnp.exp(s - m)
    kf = topk_ref[...]

    def body1(i, t):
        cand = t + jnp.left_shift(1, 29 - i)
        candf = jax.lax.bitcast_convert_type(cand, jnp.float32)
        cnt = _rsum(jnp.where(e >= candf, 1.0, 0.0))
        return jnp.where(cnt >= kf, cand, t)

    t1 = jax.lax.fori_loop(0, 30, body1, jnp.zeros((_ROWS, 1), jnp.int32))
    t1f = jax.lax.bitcast_convert_type(t1, jnp.float32)

    e1 = jnp.where(e >= t1f, e, 0.0)
    p_target = topp_ref[...] * _rsum(e1)

    def body2(i, t):
        cand = t + jnp.left_shift(1, 29 - i)
        candf = jax.lax.bitcast_convert_type(cand, jnp.float32)
        ssum = _rsum(jnp.where(e1 > candf, e1, 0.0))
        return jnp.where(ssum >= p_target, cand, t)

    tlf = jax.lax.fori_loop(0, 30, body2, jnp.zeros((_ROWS, 1), jnp.int32))
    t2f = jax.lax.bitcast_convert_type(tlf + 1, jnp.float32)
    thresh = jnp.maximum(t2f, minp_ref[...])

    z = jnp.where(e1 >= thresh, s + g_ref[...], -jnp.inf)
    zmax = _rmax(z)
    iota = jax.lax.broadcasted_iota(jnp.int32, z.shape, 1)
    out_ref[...] = _rmin(jnp.where(z == zmax, iota, _V))


@jax.jit
def kernel(logits, temperatures, top_ks, top_ps, min_ps):
    g = jax.random.gumbel(jax.random.key(42), (_B, _V), jnp.float32)
    pad = ((0, 0), (0, _VP - _V))
    logits_p = jnp.pad(logits, pad, constant_values=-jnp.inf)
    g_p = jnp.pad(g, pad)
    row = lambda x: x.reshape(_B, 1)
    grid = (_B // _ROWS,)
    wide = pl.BlockSpec((_ROWS, _VP), lambda i: (i, 0))
    slim = pl.BlockSpec((_ROWS, 1), lambda i: (i, 0))
    tokens = pl.pallas_call(
        _sampler_kernel,
        grid=grid,
        in_specs=[wide, wide, slim, slim, slim, slim],
        out_specs=slim,
        out_shape=jax.ShapeDtypeStruct((_B, 1), jnp.int32),
        compiler_params=pltpu.CompilerParams(
            dimension_semantics=("parallel",)),
    )(logits_p, g_p, row(temperatures), row(top_ks.astype(jnp.float32)),
      row(top_ps), row(min_ps))
    return tokens.reshape(_B)


# speedup vs baseline: 25.2864x; 1.0459x over previous
import functools

import jax
import jax.numpy as jnp
from jax.experimental import pallas as pl
from jax.experimental.pallas import tpu as pltpu

_B = 128
_V = 100000
_NS = 16
_SL = 6272
_ROWS = 8


def _sliced(x, red, comb):
    parts = [red(x[:, j * _SL:min((j + 1) * _SL, _V)], axis=-1,
                 keepdims=True) for j in range(_NS)]
    while len(parts) > 1:
        parts = [comb(parts[i], parts[i + 1])
                 for i in range(0, len(parts), 2)]
    return parts[0]


def _rsum(x):
    return _sliced(x, jnp.sum, jnp.add)


def _rmax(x):
    return _sliced(x, jnp.max, jnp.maximum)


def _rmin(x):
    return _sliced(x, jnp.min, jnp.minimum)


def _sampler_kernel(logits_ref, g_ref, temp_ref, topk_ref, topp_ref,
                    minp_ref, out_ref):
    s = logits_ref[...] / temp_ref[...]
    m = _rmax(s)
    e = jnp.exp(s - m)
    kf = topk_ref[...]

    def body1(i, t):
        cand = t + jnp.left_shift(1, 29 - i)
        candf = jax.lax.bitcast_convert_type(cand, jnp.float32)
        cnt = _rsum(jnp.where(e >= candf, 1.0, 0.0))
        return jnp.where(cnt >= kf, cand, t)

    t1 = jax.lax.fori_loop(0, 30, body1, jnp.zeros((_ROWS, 1), jnp.int32))
    t1f = jax.lax.bitcast_convert_type(t1, jnp.float32)

    e1 = jnp.where(e >= t1f, e, 0.0)
    p_target = topp_ref[...] * _rsum(e1)

    def body2(i, t):
        cand = t + jnp.left_shift(1, 29 - i)
        candf = jax.lax.bitcast_convert_type(cand, jnp.float32)
        ssum = _rsum(jnp.where(e1 > candf, e1, 0.0))
        return jnp.where(ssum >= p_target, cand, t)

    tlf = jax.lax.fori_loop(0, 30, body2, jnp.zeros((_ROWS, 1), jnp.int32))
    t2f = jax.lax.bitcast_convert_type(tlf + 1, jnp.float32)
    thresh = jnp.maximum(t2f, minp_ref[...])

    z = jnp.where(e1 >= thresh, s + g_ref[...], -jnp.inf)
    zmax = _rmax(z)
    iota = jax.lax.broadcasted_iota(jnp.int32, z.shape, 1)
    out_ref[...] = _rmin(jnp.where(z == zmax, iota, _V))


@jax.jit
def kernel(logits, temperatures, top_ks, top_ps, min_ps):
    g = jax.random.gumbel(jax.random.key(42), (_B, _V), jnp.float32)
    row = lambda x: x.reshape(_B, 1)
    grid = (_B // _ROWS,)
    wide = pl.BlockSpec((_ROWS, _V), lambda i: (i, 0))
    slim = pl.BlockSpec((_ROWS, 1), lambda i: (i, 0))
    tokens = pl.pallas_call(
        _sampler_kernel,
        grid=grid,
        in_specs=[wide, wide, slim, slim, slim, slim],
        out_specs=slim,
        out_shape=jax.ShapeDtypeStruct((_B, 1), jnp.int32),
        compiler_params=pltpu.CompilerParams(
            dimension_semantics=("parallel",)),
    )(logits, g, row(temperatures), row(top_ks.astype(jnp.float32)),
      row(top_ps), row(min_ps))
    return tokens.reshape(_B)
